# baseline (speedup 1.0000x reference)
import jax
import jax.numpy as jnp
from jax.experimental import pallas as pl


def _copy_body(in_ref, out_ref):
    out_ref[...] = in_ref[...]


def kernel(x):
    bsz, s, d = x.shape
    return pl.pallas_call(
        _copy_body,
        grid=(bsz,),
        in_specs=[pl.BlockSpec((1, s, d), lambda i: ((i + 1) % bsz, 0, 0))],
        out_specs=pl.BlockSpec((1, s, d), lambda i: (i, 0, 0)),
        out_shape=jax.ShapeDtypeStruct(x.shape, x.dtype),
    )(x)
